# baseline (speedup 1.0000x reference)
import functools

import jax
import jax.numpy as jnp
from jax import lax
from jax.experimental import pallas as pl
from jax.experimental.pallas import tpu as pltpu
from jax.experimental.pallas import tpu_sc as plsc


def _topk_body(k, chunk, nodes_ref, w_ref, idxn_ref):
    b = pl.program_id(0)
    N = nodes_ref.shape[1]
    x = nodes_ref[0]
    w = w_ref[...]
    s_row = lax.dot_general(w, x, (((0,), (1,)), ((), ())),
                            preferred_element_type=jnp.float32)
    s_col = jnp.transpose(s_row, (1, 0))
    r_iota = lax.broadcasted_iota(jnp.int32, (1, k), 1)
    j_iota = lax.broadcasted_iota(jnp.int32, (chunk, N), 1)
    idx_acc = jnp.zeros((1, k), dtype=jnp.int32)
    for ci in range(0, N, chunk):
        sc = s_col[ci:ci + chunk, :]
        i_col = ci + lax.broadcasted_iota(jnp.int32, (chunk, 1), 0)
        before = (s_row > sc) | ((s_row == sc) & (j_iota < i_col))
        rank = jnp.sum(jnp.where(before, 1.0, 0.0), axis=1,
                       keepdims=True).astype(jnp.int32)
        onehot = rank == r_iota
        contrib = jnp.where(onehot, i_col, 0)
        idx_acc = idx_acc + jnp.sum(contrib, axis=0, keepdims=True)
    idxn_ref[0] = idx_acc + b * N


def _topk_call(nodes, W, k, chunk=256):
    B, N, C = nodes.shape
    body = functools.partial(_topk_body, k, chunk)
    return pl.pallas_call(
        body,
        grid=(B,),
        in_specs=[
            pl.BlockSpec((1, N, C), lambda b: (b, 0, 0)),
            pl.BlockSpec((C, 1), lambda b: (0, 0)),
        ],
        out_specs=pl.BlockSpec((1, 1, k), lambda b: (b, 0, 0)),
        out_shape=jax.ShapeDtypeStruct((B, 1, k), jnp.int32),
    )(nodes, W)


def _gather_call(idxn, nodes_flat, adj_flat, k, rows_per_w, ch):
    BK = idxn.shape[0]
    C = nodes_flat.shape[1]
    N = adj_flat.shape[1]
    info = plsc.get_sparse_core_info()
    nc = info.num_cores

    mesh = plsc.VectorSubcoreMesh(core_axis_name="c", subcore_axis_name="s")

    @functools.partial(
        pl.kernel,
        mesh=mesh,
        out_type=[
            jax.ShapeDtypeStruct((BK, C), jnp.float32),
            jax.ShapeDtypeStruct((BK, k), jnp.float32),
        ],
        scratch_types=[
            pltpu.VMEM((ch,), jnp.int32),
            pltpu.VMEM((ch,), jnp.int32),
            pltpu.VMEM((ch, C), jnp.float32),
            pltpu.VMEM((ch, C), jnp.float32),
            pltpu.VMEM((ch, N), jnp.float32),
            pltpu.VMEM((ch, N), jnp.float32),
            pltpu.SemaphoreType.DMA,
            pltpu.SemaphoreType.DMA,
            pltpu.SemaphoreType.DMA,
            pltpu.SemaphoreType.DMA,
            pltpu.SemaphoreType.DMA,
            pltpu.SemaphoreType.DMA,
            pltpu.SemaphoreType.DMA,
            pltpu.SemaphoreType.DMA,
        ],
    )
    def run(idxn_hbm, nodes_hbm, adj_hbm, outn_hbm, outa_hbm,
            idx0, idx1, nbuf0, nbuf1, abuf0, abuf1,
            gn0, gn1, ga0, ga1, sn0, sn1, sa0, sa1):
        wid = lax.axis_index("s") * nc + lax.axis_index("c")
        base = wid * rows_per_w
        idxs = [idx0, idx1]
        nbufs, abufs = [nbuf0, nbuf1], [abuf0, abuf1]
        gns, gas = [gn0, gn1], [ga0, ga1]
        sns, sas = [sn0, sn1], [sa0, sa1]
        nch = rows_per_w // ch
        g_n = [None] * nch
        g_a = [None] * nch
        s_n = [None] * nch
        s_a = [None] * nch

        def start_gather(c):
            bi = c % 2
            pltpu.sync_copy(idxn_hbm.at[pl.ds(base + c * ch, ch)], idxs[bi])
            g_n[c] = pltpu.async_copy(nodes_hbm.at[idxs[bi]], nbufs[bi],
                                      gns[bi])
            g_a[c] = pltpu.async_copy(adj_hbm.at[idxs[bi]], abufs[bi],
                                      gas[bi])

        start_gather(0)
        for c in range(nch):
            bi = c % 2
            g_n[c].wait()
            g_a[c].wait()
            off = base + c * ch
            s_n[c] = pltpu.async_copy(nbufs[bi], outn_hbm.at[pl.ds(off, ch)],
                                      sns[bi])
            s_a[c] = pltpu.async_copy(abufs[bi].at[:, pl.ds(0, k)],
                                      outa_hbm.at[pl.ds(off, ch)], sas[bi])
            if c + 1 < nch:
                if c >= 1:
                    s_n[c - 1].wait()
                    s_a[c - 1].wait()
                start_gather(c + 1)
        s_n[nch - 2].wait()
        s_a[nch - 2].wait()
        s_n[nch - 1].wait()
        s_a[nch - 1].wait()

    return run(idxn, nodes_flat, adj_flat)


def kernel(nodes, adj_mat, W, b):
    B, N, C = nodes.shape
    k = N // 2
    idxn = _topk_call(nodes, W, k)
    nodes_flat = nodes.reshape(B * N, C)
    adj_flat = adj_mat.reshape(B * N, N)
    nw = 32
    out_n, out_a = _gather_call(
        idxn.reshape(B * k), nodes_flat, adj_flat, k,
        rows_per_w=(B * k) // nw, ch=16)
    return out_n.reshape(B, k, C), out_a.reshape(B, k, k)


# speedup vs baseline: 1.1962x; 1.1962x over previous
import functools

import jax
import jax.numpy as jnp
from jax import lax
from jax.experimental import pallas as pl
from jax.experimental.pallas import tpu as pltpu
from jax.experimental.pallas import tpu_sc as plsc


def _topk_body(k, chunk, nodes_ref, w_ref, idxn_ref):
    b = pl.program_id(0)
    N = nodes_ref.shape[1]
    x = nodes_ref[0]
    w = w_ref[...]
    s_row = lax.dot_general(w, x, (((0,), (1,)), ((), ())),
                            preferred_element_type=jnp.float32)
    s_col = jnp.transpose(s_row, (1, 0))
    r_iota = lax.broadcasted_iota(jnp.int32, (1, k), 1)
    j_iota = lax.broadcasted_iota(jnp.int32, (chunk, N), 1)
    idx_acc = jnp.zeros((1, k), dtype=jnp.int32)
    for ci in range(0, N, chunk):
        sc = s_col[ci:ci + chunk, :]
        i_col = ci + lax.broadcasted_iota(jnp.int32, (chunk, 1), 0)
        before = (s_row > sc) | ((s_row == sc) & (j_iota < i_col))
        rank = jnp.sum(jnp.where(before, 1.0, 0.0), axis=1,
                       keepdims=True).astype(jnp.int32)
        onehot = rank == r_iota
        contrib = jnp.where(onehot, i_col, 0)
        idx_acc = idx_acc + jnp.sum(contrib, axis=0, keepdims=True)
    idxn_ref[0] = idx_acc + b * N


def _topk_call(nodes, W, k, chunk=256):
    B, N, C = nodes.shape
    body = functools.partial(_topk_body, k, chunk)
    return pl.pallas_call(
        body,
        grid=(B,),
        in_specs=[
            pl.BlockSpec((1, N, C), lambda b: (b, 0, 0)),
            pl.BlockSpec((C, 1), lambda b: (0, 0)),
        ],
        out_specs=pl.BlockSpec((1, 1, k), lambda b: (b, 0, 0)),
        out_shape=jax.ShapeDtypeStruct((B, 1, k), jnp.int32),
    )(nodes, W)


def _gather_call(idxn, nodes_flat, adj_flat, k, rows_per_w, ch):
    BK = idxn.shape[0]
    C = nodes_flat.shape[1]
    N = adj_flat.shape[1]
    info = plsc.get_sparse_core_info()
    nc = info.num_cores

    mesh = plsc.VectorSubcoreMesh(core_axis_name="c", subcore_axis_name="s")

    @functools.partial(
        pl.kernel,
        mesh=mesh,
        out_type=[
            jax.ShapeDtypeStruct((BK, C), jnp.float32),
            jax.ShapeDtypeStruct((BK, k), jnp.float32),
        ],
        scratch_types=[
            pltpu.VMEM((ch,), jnp.int32),
            pltpu.VMEM((ch,), jnp.int32),
            pltpu.VMEM((ch, C), jnp.float32),
            pltpu.VMEM((ch, C), jnp.float32),
            pltpu.VMEM((ch, k), jnp.float32),
            pltpu.VMEM((ch, k), jnp.float32),
            pltpu.SemaphoreType.DMA,
            pltpu.SemaphoreType.DMA,
            pltpu.SemaphoreType.DMA,
            pltpu.SemaphoreType.DMA,
            pltpu.SemaphoreType.DMA,
            pltpu.SemaphoreType.DMA,
            pltpu.SemaphoreType.DMA,
            pltpu.SemaphoreType.DMA,
        ],
    )
    def run(idxn_hbm, nodes_hbm, adj_hbm, outn_hbm, outa_hbm,
            idx0, idx1, nbuf0, nbuf1, abuf0, abuf1,
            gn0, gn1, ga0, ga1, sn0, sn1, sa0, sa1):
        wid = lax.axis_index("s") * nc + lax.axis_index("c")
        base = wid * rows_per_w
        idxs = [idx0, idx1]
        nbufs, abufs = [nbuf0, nbuf1], [abuf0, abuf1]
        gns, gas = [gn0, gn1], [ga0, ga1]
        sns, sas = [sn0, sn1], [sa0, sa1]
        nch = rows_per_w // ch
        g_n = [None] * nch
        g_a = [None] * nch
        s_n = [None] * nch
        s_a = [None] * nch

        def start_gather(c):
            bi = c % 2
            pltpu.sync_copy(idxn_hbm.at[pl.ds(base + c * ch, ch)], idxs[bi])
            g_n[c] = pltpu.async_copy(nodes_hbm.at[idxs[bi]], nbufs[bi],
                                      gns[bi])
            g_a[c] = pltpu.async_copy(adj_hbm.at[idxs[bi], pl.ds(0, k)],
                                      abufs[bi], gas[bi])

        start_gather(0)
        for c in range(nch):
            bi = c % 2
            g_n[c].wait()
            g_a[c].wait()
            off = base + c * ch
            s_n[c] = pltpu.async_copy(nbufs[bi], outn_hbm.at[pl.ds(off, ch)],
                                      sns[bi])
            s_a[c] = pltpu.async_copy(abufs[bi],
                                      outa_hbm.at[pl.ds(off, ch)], sas[bi])
            if c + 1 < nch:
                if c >= 1:
                    s_n[c - 1].wait()
                    s_a[c - 1].wait()
                start_gather(c + 1)
        s_n[nch - 2].wait()
        s_a[nch - 2].wait()
        s_n[nch - 1].wait()
        s_a[nch - 1].wait()

    return run(idxn, nodes_flat, adj_flat)


def kernel(nodes, adj_mat, W, b):
    B, N, C = nodes.shape
    k = N // 2
    idxn = _topk_call(nodes, W, k)
    nodes_flat = nodes.reshape(B * N, C)
    adj_flat = adj_mat.reshape(B * N, N)
    nw = 32
    out_n, out_a = _gather_call(
        idxn.reshape(B * k), nodes_flat, adj_flat, k,
        rows_per_w=(B * k) // nw, ch=32)
    return out_n.reshape(B, k, C), out_a.reshape(B, k, k)


# speedup vs baseline: 1.2228x; 1.0223x over previous
import functools

import jax
import jax.numpy as jnp
from jax import lax
from jax.experimental import pallas as pl
from jax.experimental.pallas import tpu as pltpu
from jax.experimental.pallas import tpu_sc as plsc


def _topk_body(k, chunk, nodes_ref, w_ref, idxn_ref):
    b = pl.program_id(0)
    N = nodes_ref.shape[1]
    x = nodes_ref[0]
    w = w_ref[...]
    s_row = lax.dot_general(w, x, (((0,), (1,)), ((), ())),
                            preferred_element_type=jnp.float32)
    s_col = jnp.transpose(s_row, (1, 0))
    r_iota = lax.broadcasted_iota(jnp.int32, (1, k), 1)
    j_iota = lax.broadcasted_iota(jnp.int32, (chunk, N), 1)
    idx_acc = jnp.zeros((1, k), dtype=jnp.int32)
    for ci in range(0, N, chunk):
        sc = s_col[ci:ci + chunk, :]
        i_col = ci + lax.broadcasted_iota(jnp.int32, (chunk, 1), 0)
        before = (s_row > sc) | ((s_row == sc) & (j_iota < i_col))
        rank = jnp.sum(jnp.where(before, 1.0, 0.0), axis=1,
                       keepdims=True).astype(jnp.int32)
        onehot = rank == r_iota
        contrib = jnp.where(onehot, i_col, 0)
        idx_acc = idx_acc + jnp.sum(contrib, axis=0, keepdims=True)
    idxn_ref[0] = idx_acc + b * N


def _topk_call(nodes, W, k, chunk=256):
    B, N, C = nodes.shape
    body = functools.partial(_topk_body, k, chunk)
    return pl.pallas_call(
        body,
        grid=(B,),
        in_specs=[
            pl.BlockSpec((1, N, C), lambda b: (b, 0, 0)),
            pl.BlockSpec((C, 1), lambda b: (0, 0)),
        ],
        out_specs=pl.BlockSpec((1, 1, k), lambda b: (b, 0, 0)),
        out_shape=jax.ShapeDtypeStruct((B, 1, k), jnp.int32),
    )(nodes, W)


def _gather_call(idxn, nodes_flat, adj_flat, k, rows_per_w, ch):
    BK = idxn.shape[0]
    C = nodes_flat.shape[1]
    N = adj_flat.shape[1]
    info = plsc.get_sparse_core_info()
    nc = info.num_cores

    mesh = plsc.VectorSubcoreMesh(core_axis_name="c", subcore_axis_name="s")

    @functools.partial(
        pl.kernel,
        mesh=mesh,
        out_type=[
            jax.ShapeDtypeStruct((BK, C), jnp.float32),
            jax.ShapeDtypeStruct((BK, k), jnp.float32),
        ],
        scratch_types=[
            pltpu.VMEM((rows_per_w,), jnp.int32),
            pltpu.VMEM((ch, C), jnp.float32),
            pltpu.VMEM((ch, C), jnp.float32),
            pltpu.VMEM((ch, k), jnp.float32),
            pltpu.VMEM((ch, k), jnp.float32),
            pltpu.SemaphoreType.DMA,
            pltpu.SemaphoreType.DMA,
            pltpu.SemaphoreType.DMA,
            pltpu.SemaphoreType.DMA,
            pltpu.SemaphoreType.DMA,
            pltpu.SemaphoreType.DMA,
            pltpu.SemaphoreType.DMA,
            pltpu.SemaphoreType.DMA,
        ],
    )
    def run(idxn_hbm, nodes_hbm, adj_hbm, outn_hbm, outa_hbm,
            idx_v, nbuf0, nbuf1, abuf0, abuf1,
            gn0, gn1, ga0, ga1, sn0, sn1, sa0, sa1):
        wid = lax.axis_index("s") * nc + lax.axis_index("c")
        base = wid * rows_per_w
        pltpu.sync_copy(idxn_hbm.at[pl.ds(base, rows_per_w)], idx_v)
        nbufs, abufs = [nbuf0, nbuf1], [abuf0, abuf1]
        gns, gas = [gn0, gn1], [ga0, ga1]
        sns, sas = [sn0, sn1], [sa0, sa1]
        nch = rows_per_w // ch
        g_n = [None] * nch
        g_a = [None] * nch
        s_n = [None] * nch
        s_a = [None] * nch

        def start_gather(c):
            bi = c % 2
            ids = idx_v.at[pl.ds(c * ch, ch)]
            g_n[c] = pltpu.async_copy(nodes_hbm.at[ids], nbufs[bi],
                                      gns[bi])
            g_a[c] = pltpu.async_copy(adj_hbm.at[ids, pl.ds(0, k)],
                                      abufs[bi], gas[bi])

        start_gather(0)
        for c in range(nch):
            bi = c % 2
            g_n[c].wait()
            g_a[c].wait()
            off = base + c * ch
            s_n[c] = pltpu.async_copy(nbufs[bi], outn_hbm.at[pl.ds(off, ch)],
                                      sns[bi])
            s_a[c] = pltpu.async_copy(abufs[bi],
                                      outa_hbm.at[pl.ds(off, ch)], sas[bi])
            if c + 1 < nch:
                if c >= 1:
                    s_n[c - 1].wait()
                    s_a[c - 1].wait()
                start_gather(c + 1)
        s_n[nch - 2].wait()
        s_a[nch - 2].wait()
        s_n[nch - 1].wait()
        s_a[nch - 1].wait()

    return run(idxn, nodes_flat, adj_flat)


def kernel(nodes, adj_mat, W, b):
    B, N, C = nodes.shape
    k = N // 2
    idxn = _topk_call(nodes, W, k)
    nodes_flat = nodes.reshape(B * N, C)
    adj_flat = adj_mat.reshape(B * N, N)
    nw = 32
    out_n, out_a = _gather_call(
        idxn.reshape(B * k), nodes_flat, adj_flat, k,
        rows_per_w=(B * k) // nw, ch=32)
    return out_n.reshape(B, k, C), out_a.reshape(B, k, k)


# speedup vs baseline: 1.2862x; 1.0518x over previous
import functools

import jax
import jax.numpy as jnp
from jax import lax
from jax.experimental import pallas as pl
from jax.experimental.pallas import tpu as pltpu
from jax.experimental.pallas import tpu_sc as plsc


def _topk_body(k, chunk, nodes_ref, w_ref, idxn_ref):
    b = pl.program_id(0)
    N = nodes_ref.shape[1]
    x = nodes_ref[0]
    w = w_ref[...]
    s_row = lax.dot_general(w, x, (((0,), (1,)), ((), ())),
                            preferred_element_type=jnp.float32)
    s_col = jnp.transpose(s_row, (1, 0))
    r_iota = lax.broadcasted_iota(jnp.int32, (1, k), 1)
    j_iota = lax.broadcasted_iota(jnp.int32, (chunk, N), 1)
    idx_acc = jnp.zeros((1, k), dtype=jnp.int32)
    for ci in range(0, N, chunk):
        sc = s_col[ci:ci + chunk, :]
        i_col = ci + lax.broadcasted_iota(jnp.int32, (chunk, 1), 0)
        before = (s_row > sc) | ((s_row == sc) & (j_iota < i_col))
        rank = jnp.sum(jnp.where(before, 1.0, 0.0), axis=1,
                       keepdims=True).astype(jnp.int32)
        onehot = rank == r_iota
        contrib = jnp.where(onehot, i_col, 0)
        idx_acc = idx_acc + jnp.sum(contrib, axis=0, keepdims=True)
    idxn_ref[0] = idx_acc + b * N


def _topk_call(nodes, W, k, chunk=256):
    B, N, C = nodes.shape
    body = functools.partial(_topk_body, k, chunk)
    return pl.pallas_call(
        body,
        grid=(B,),
        in_specs=[
            pl.BlockSpec((1, N, C), lambda b: (b, 0, 0)),
            pl.BlockSpec((C, 1), lambda b: (0, 0)),
        ],
        out_specs=pl.BlockSpec((1, 1, k), lambda b: (b, 0, 0)),
        out_shape=jax.ShapeDtypeStruct((B, 1, k), jnp.int32),
    )(nodes, W)


def _gather_call(idxn, nodes_flat, adj_flat, k, rows_per_w, ch):
    BK = idxn.shape[0]
    C = nodes_flat.shape[1]
    N = adj_flat.shape[1]
    info = plsc.get_sparse_core_info()
    nc = info.num_cores

    mesh = plsc.VectorSubcoreMesh(core_axis_name="c", subcore_axis_name="s")

    @functools.partial(
        pl.kernel,
        mesh=mesh,
        out_type=[
            jax.ShapeDtypeStruct((BK, C), jnp.float32),
            jax.ShapeDtypeStruct((BK, k), jnp.float32),
        ],
        scratch_types=(
            [pltpu.VMEM((rows_per_w,), jnp.int32)]
            + [pltpu.VMEM((ch, C), jnp.float32)] * 3
            + [pltpu.VMEM((ch, k), jnp.float32)] * 3
            + [pltpu.SemaphoreType.DMA] * 12
        ),
    )
    def run(idxn_hbm, nodes_hbm, adj_hbm, outn_hbm, outa_hbm,
            idx_v, nb0, nb1, nb2, ab0, ab1, ab2,
            gn0, gn1, gn2, ga0, ga1, ga2,
            sn0, sn1, sn2, sa0, sa1, sa2):
        wid = lax.axis_index("s") * nc + lax.axis_index("c")
        base = wid * rows_per_w
        pltpu.sync_copy(idxn_hbm.at[pl.ds(base, rows_per_w)], idx_v)
        NB = 3
        nbufs, abufs = [nb0, nb1, nb2], [ab0, ab1, ab2]
        gns, gas = [gn0, gn1, gn2], [ga0, ga1, ga2]
        sns, sas = [sn0, sn1, sn2], [sa0, sa1, sa2]
        nch = rows_per_w // ch
        g_n = [None] * nch
        g_a = [None] * nch
        s_n = [None] * nch
        s_a = [None] * nch

        def start_gather(c):
            bi = c % NB
            ids = idx_v.at[pl.ds(c * ch, ch)]
            g_n[c] = pltpu.async_copy(nodes_hbm.at[ids], nbufs[bi],
                                      gns[bi])
            g_a[c] = pltpu.async_copy(adj_hbm.at[ids, pl.ds(0, k)],
                                      abufs[bi], gas[bi])

        for c0 in range(min(NB - 1, nch)):
            start_gather(c0)
        for c in range(nch):
            bi = c % NB
            nxt = c + NB - 1
            if nxt < nch:
                if c >= 1:
                    s_n[c - 1].wait()
                    s_a[c - 1].wait()
                start_gather(nxt)
            g_n[c].wait()
            g_a[c].wait()
            off = base + c * ch
            s_n[c] = pltpu.async_copy(nbufs[bi], outn_hbm.at[pl.ds(off, ch)],
                                      sns[bi])
            s_a[c] = pltpu.async_copy(abufs[bi],
                                      outa_hbm.at[pl.ds(off, ch)], sas[bi])
        for c in range(max(0, nch - NB), nch):
            s_n[c].wait()
            s_a[c].wait()

    return run(idxn, nodes_flat, adj_flat)


def kernel(nodes, adj_mat, W, b):
    B, N, C = nodes.shape
    k = N // 2
    idxn = _topk_call(nodes, W, k)
    nodes_flat = nodes.reshape(B * N, C)
    adj_flat = adj_mat.reshape(B * N, N)
    nw = 32
    out_n, out_a = _gather_call(
        idxn.reshape(B * k), nodes_flat, adj_flat, k,
        rows_per_w=(B * k) // nw, ch=32)
    return out_n.reshape(B, k, C), out_a.reshape(B, k, k)
